# baseline (speedup 1.0000x reference)
import functools

import jax
import jax.numpy as jnp
from jax import lax
from jax.experimental import pallas as pl
from jax.experimental.pallas import tpu as pltpu
from jax.experimental.pallas import tpu_sc as plsc

_SUM_OF_FEATURES = 100000
_LINPUT = 256
_NUM_LS_BUCKETS = 60
_MAX_PLY = 60
_BUCKET_SIZE = _MAX_PLY // _NUM_LS_BUCKETS
_B = 16384
_N_ACTIVE = 26

_NC = 2
_NS = 16
_NW = _NC * _NS
_BPW = _B // _NW
_C = 4
_ROWS = _C * _N_ACTIVE
_NCHUNK = _BPW // _C
_NSLOT = 4
_L = 16
_K2 = _LINPUT // 32

_mesh = plsc.VectorSubcoreMesh(
    core_axis_name="c", subcore_axis_name="s",
    num_cores=_NC, num_subcores=_NS)


@functools.partial(
    pl.kernel,
    out_type=jax.ShapeDtypeStruct((_B,), jnp.float32),
    mesh=_mesh,
    scratch_types=[
        pltpu.VMEM((_NCHUNK, _ROWS), jnp.int32),
        pltpu.VMEM((_BPW,), jnp.int32),
        pltpu.VMEM((_NUM_LS_BUCKETS * _LINPUT,), jnp.float32),
        pltpu.VMEM((_LINPUT,), jnp.float32),
        pltpu.VMEM((_NUM_LS_BUCKETS,), jnp.float32),
        pltpu.VMEM((_NSLOT, _ROWS, _LINPUT), jnp.bfloat16),
        pltpu.VMEM((_BPW + _L,), jnp.float32),
        pltpu.SemaphoreType.DMA,
    ],
    compiler_params=pltpu.CompilerParams(
        needs_layout_passes=False, use_tc_tiling_on_sc=False),
)
def _sc_forward(fi_hbm, ply_hbm, table_hbm, ib_hbm, w_hbm, ob_hbm, out_hbm,
                fi_v, ply_v, w_v, ib_v, ob_v, rows_v, out_v, sem):
    wid = lax.axis_index("s") * _NC + lax.axis_index("c")

    pltpu.sync_copy(fi_hbm.at[wid], fi_v)
    pltpu.sync_copy(ply_hbm.at[pl.ds(wid * _BPW, _BPW)], ply_v)
    pltpu.sync_copy(w_hbm, w_v)
    pltpu.sync_copy(ib_hbm, ib_v)
    pltpu.sync_copy(ob_hbm, ob_v)

    lane = lax.iota(jnp.int32, _L)

    def issue(chunk, slot):
        pltpu.async_copy(table_hbm.at[fi_v.at[chunk]], rows_v.at[slot], sem)

    def wait(chunk, slot):
        pltpu.make_async_copy(
            table_hbm.at[fi_v.at[chunk]], rows_v.at[slot], sem).wait()

    for c in range(_NSLOT - 1):
        issue(c, c)

    @pl.loop(0, _NCHUNK)
    def _chunk_body(chunk):
        slot = lax.rem(chunk, _NSLOT)
        wait(chunk, slot)

        @pl.when(chunk + _NSLOT - 1 < _NCHUNK)
        def _():
            issue(chunk + _NSLOT - 1, lax.rem(chunk + _NSLOT - 1, _NSLOT))

        buckets = []
        for b in range(_C):
            bidx = jnp.full((_L,), 0, jnp.int32) + (chunk * _C + b)
            plyb = plsc.load_gather(ply_v, [bidx])
            buckets.append(jnp.clip(plyb // _BUCKET_SIZE, 0, _NUM_LS_BUCKETS - 1))

        init = tuple(jnp.zeros((_L,), jnp.float32) for _ in range(_C))

        @pl.loop(0, _K2, init_carry=init)
        def partials(k2, carry):
            col = pl.ds(k2 * 32, 32)
            bias_a = ib_v[pl.ds(k2 * 32, _L)]
            bias_b = ib_v[pl.ds(k2 * 32 + _L, _L)]
            new = []
            for b in range(_C):
                base = b * _N_ACTIVE
                pairs = [rows_v[slot, base + 2 * r, col]
                         + rows_v[slot, base + 2 * r + 1, col]
                         for r in range(_N_ACTIVE // 2)]
                fa, fb = [], []
                for p in pairs:
                    a_, b_ = plsc.unpack(p, format=plsc.PackFormat.INTERLEAVED)
                    fa.append(a_)
                    fb.append(b_)
                acc = []
                for vals13, bias in ((fa, bias_a), (fb, bias_b)):
                    chains = vals13[:4]
                    for r in range(4, len(vals13)):
                        chains[r % 4] = chains[r % 4] + vals13[r]
                    a = (chains[0] + chains[1]) + (chains[2] + chains[3])
                    a = a + bias
                    a = jnp.clip(a, 0.0, 1.0)
                    acc.append(a * a * jnp.float32(255.0 / 256.0))
                wbase = buckets[b] * _LINPUT + (k2 * 32 + lane)
                wa = plsc.load_gather(w_v, [wbase])
                wb = plsc.load_gather(w_v, [wbase + _L])
                new.append(carry[b] + acc[0] * wa + acc[1] * wb)
            return tuple(new)

        vals = jnp.zeros((_L,), jnp.float32)
        for b in range(_C):
            rsum = jnp.sum(partials[b])
            ob = plsc.load_gather(ob_v, [buckets[b]])
            vals = vals + jnp.where(lane == b, rsum + ob, 0.0)

        out_v[pl.ds(chunk * _C, _L)] = vals

    pltpu.sync_copy(out_v.at[pl.ds(0, _BPW)],
                    out_hbm.at[pl.ds(wid * _BPW, _BPW)])


def kernel(feature_indices, mobility, ply, input_weight, input_bias,
           output_weight, output_bias):
    del mobility
    fi = feature_indices.reshape(_NW, _NCHUNK, _ROWS)
    tbl16 = input_weight.astype(jnp.bfloat16)
    ib2 = input_bias.reshape(_K2, _L, 2).transpose(0, 2, 1).reshape(-1)
    w2 = (output_weight.reshape(_NUM_LS_BUCKETS, _K2, _L, 2)
          .transpose(0, 1, 3, 2).reshape(-1))
    out = _sc_forward(fi, ply.astype(jnp.int32), tbl16, ib2, w2, output_bias)
    return out.reshape(_B, 1)


# speedup vs baseline: 1.6087x; 1.6087x over previous
import functools

import jax
import jax.numpy as jnp
from jax import lax
from jax.experimental import pallas as pl
from jax.experimental.pallas import tpu as pltpu
from jax.experimental.pallas import tpu_sc as plsc

_SUM_OF_FEATURES = 100000
_LINPUT = 256
_NUM_LS_BUCKETS = 60
_MAX_PLY = 60
_BUCKET_SIZE = _MAX_PLY // _NUM_LS_BUCKETS
_B = 16384
_N_ACTIVE = 26

_NC = 2
_NS = 16
_NW = _NC * _NS
_BPW = _B // _NW
_C = 4
_ROWS = _C * _N_ACTIVE
_NCHUNK = _BPW // _C
_L = 16
_KCH = _LINPUT // _L

_LN = 128
_NLINES = _SUM_OF_FEATURES * 2

_mesh = plsc.VectorSubcoreMesh(
    core_axis_name="c", subcore_axis_name="s",
    num_cores=_NC, num_subcores=_NS)


@functools.partial(
    pl.kernel,
    out_type=jax.ShapeDtypeStruct((_B,), jnp.float32),
    mesh=_mesh,
    scratch_types=[
        pltpu.VMEM((_NCHUNK, 2, _ROWS), jnp.int32),
        pltpu.VMEM((_BPW,), jnp.int32),
        pltpu.VMEM((_NUM_LS_BUCKETS * _LINPUT,), jnp.float32),
        pltpu.VMEM((_LINPUT,), jnp.float32),
        pltpu.VMEM((_NUM_LS_BUCKETS,), jnp.float32),
        pltpu.VMEM((3, 2 * _ROWS, _LN), jnp.float32),
        pltpu.VMEM((_BPW + _L,), jnp.float32),
        pltpu.SemaphoreType.DMA,
    ],
    compiler_params=pltpu.CompilerParams(
        needs_layout_passes=False, use_tc_tiling_on_sc=False),
)
def _sc_forward(fi_hbm, ply_hbm, table_hbm, ib_hbm, w_hbm, ob_hbm, out_hbm,
                fi_v, ply_v, w_v, ib_v, ob_v, rows_v, out_v, sem):
    wid = lax.axis_index("s") * _NC + lax.axis_index("c")

    pltpu.sync_copy(fi_hbm.at[wid], fi_v)
    pltpu.sync_copy(ply_hbm.at[pl.ds(wid * _BPW, _BPW)], ply_v)
    pltpu.sync_copy(w_hbm, w_v)
    pltpu.sync_copy(ib_hbm, ib_v)
    pltpu.sync_copy(ob_hbm, ob_v)

    lane = lax.iota(jnp.int32, _L)

    def issue(chunk, slot):
        for h in range(2):
            pltpu.async_copy(table_hbm.at[fi_v.at[chunk, h]],
                             rows_v.at[slot, pl.ds(h * _ROWS, _ROWS)], sem)

    def wait(chunk, slot):
        for h in range(2):
            pltpu.make_async_copy(
                table_hbm.at[fi_v.at[chunk, h]],
                rows_v.at[slot, pl.ds(h * _ROWS, _ROWS)], sem).wait()

    issue(0, 0)
    issue(1, 1)

    @pl.loop(0, _NCHUNK)
    def _chunk_body(chunk):
        slot = lax.rem(chunk, 3)
        wait(chunk, slot)

        @pl.when(chunk + 2 < _NCHUNK)
        def _():
            issue(chunk + 2, lax.rem(chunk + 2, 3))

        buckets = []
        for b in range(_C):
            bidx = jnp.full((_L,), 0, jnp.int32) + (chunk * _C + b)
            plyb = plsc.load_gather(ply_v, [bidx])
            buckets.append(jnp.clip(plyb // _BUCKET_SIZE, 0, _NUM_LS_BUCKETS - 1))

        init = tuple(jnp.zeros((_L,), jnp.float32) for _ in range(_C))

        @pl.loop(0, _KCH, init_carry=init)
        def partials(k, carry):
            col = pl.ds(k * _L, _L)
            bias = ib_v[col]
            hbase = (k // 8) * _ROWS
            lcol = pl.ds(lax.rem(k, 8) * _L, _L)
            new = []
            for b in range(_C):
                base = b * _N_ACTIVE
                chains = [rows_v[slot, hbase + base + r, lcol]
                          for r in range(4)]
                for r in range(4, _N_ACTIVE):
                    chains[r % 4] = (chains[r % 4]
                                     + rows_v[slot, hbase + base + r, lcol])
                acc = (chains[0] + chains[1]) + (chains[2] + chains[3])
                acc = acc + bias
                acc = jnp.clip(acc, 0.0, 1.0)
                acc = acc * acc * jnp.float32(255.0 / 256.0)
                widx = buckets[b] * _LINPUT + (k * _L + lane)
                wch = plsc.load_gather(w_v, [widx])
                new.append(carry[b] + acc * wch)
            return tuple(new)

        vals = jnp.zeros((_L,), jnp.float32)
        for b in range(_C):
            rsum = jnp.sum(partials[b])
            ob = plsc.load_gather(ob_v, [buckets[b]])
            vals = vals + jnp.where(lane == b, rsum + ob, 0.0)

        out_v[pl.ds(chunk * _C, _L)] = vals

    pltpu.sync_copy(out_v.at[pl.ds(0, _BPW)],
                    out_hbm.at[pl.ds(wid * _BPW, _BPW)])


def kernel(feature_indices, mobility, ply, input_weight, input_bias,
           output_weight, output_bias):
    del mobility
    line0 = (feature_indices // 8) * 16 + (feature_indices % 8)
    l0r = line0.reshape(_NW, _NCHUNK, _ROWS)
    fi2 = jnp.stack([l0r, l0r + 8], axis=2)
    tbl = (input_weight.reshape(12500, 8, 2, _LN)
           .transpose(0, 2, 1, 3).reshape(_NLINES, _LN))
    w_flat = output_weight.reshape(-1)
    out = _sc_forward(fi2, ply.astype(jnp.int32), tbl, input_bias,
                      w_flat, output_bias)
    return out.reshape(_B, 1)


# speedup vs baseline: 1.7416x; 1.0826x over previous
import functools

import jax
import jax.numpy as jnp
from jax import lax
from jax.experimental import pallas as pl
from jax.experimental.pallas import tpu as pltpu
from jax.experimental.pallas import tpu_sc as plsc

_SUM_OF_FEATURES = 100000
_LINPUT = 256
_NUM_LS_BUCKETS = 60
_MAX_PLY = 60
_BUCKET_SIZE = _MAX_PLY // _NUM_LS_BUCKETS
_B = 16384
_N_ACTIVE = 26

_NC = 2
_NS = 16
_NW = _NC * _NS
_BPW = _B // _NW
_C = 4
_ROWS = _C * _N_ACTIVE
_RPAD = 112
_NCHUNK = _BPW // _C
_L = 16
_KCH = _LINPUT // _L

_LN = 128
_NLINES = _SUM_OF_FEATURES * 2

_mesh = plsc.VectorSubcoreMesh(
    core_axis_name="c", subcore_axis_name="s",
    num_cores=_NC, num_subcores=_NS)


@functools.partial(
    pl.kernel,
    out_type=jax.ShapeDtypeStruct((_B,), jnp.float32),
    mesh=_mesh,
    scratch_types=[
        pltpu.VMEM((_BPW, _N_ACTIVE), jnp.int32),
        pltpu.VMEM((3, 2, _RPAD), jnp.int32),
        pltpu.VMEM((_BPW,), jnp.int32),
        pltpu.VMEM((_NUM_LS_BUCKETS * _LINPUT,), jnp.float32),
        pltpu.VMEM((_LINPUT,), jnp.float32),
        pltpu.VMEM((_NUM_LS_BUCKETS,), jnp.float32),
        pltpu.VMEM((3, 2 * _ROWS, _LN), jnp.float32),
        pltpu.VMEM((_BPW + _L,), jnp.float32),
        pltpu.SemaphoreType.DMA,
    ],
    compiler_params=pltpu.CompilerParams(
        needs_layout_passes=False, use_tc_tiling_on_sc=False),
)
def _sc_forward(fi_hbm, ply_hbm, table_hbm, ib_hbm, w_hbm, ob_hbm, out_hbm,
                fi_raw_v, fi_v, ply_v, w_v, ib_v, ob_v, rows_v, out_v, sem):
    wid = lax.axis_index("s") * _NC + lax.axis_index("c")

    pltpu.sync_copy(fi_hbm.at[pl.ds(wid * _BPW, _BPW)], fi_raw_v)
    pltpu.sync_copy(ply_hbm.at[pl.ds(wid * _BPW, _BPW)], ply_v)
    pltpu.sync_copy(w_hbm, w_v)
    pltpu.sync_copy(ib_hbm, ib_v)
    pltpu.sync_copy(ob_hbm, ob_v)

    lane = lax.iota(jnp.int32, _L)

    def prep_indices(chunk):
        slot = lax.rem(chunk, 3)
        for g in range(_RPAD // _L):
            p = lane + (g * _L)
            rv = p // _N_ACTIVE
            cv = p - rv * _N_ACTIVE
            row = jnp.minimum(chunk * _C + rv, _BPW - 1)
            v = plsc.load_gather(fi_raw_v, [row, cv])
            l0 = (v >> 3) * 16 + (v & 7)
            fi_v[slot, 0, pl.ds(g * _L, _L)] = l0
            fi_v[slot, 1, pl.ds(g * _L, _L)] = l0 + 8

    def issue(chunk, slot):
        islot = lax.rem(chunk, 3)
        for h in range(2):
            pltpu.async_copy(table_hbm.at[fi_v.at[islot, h, pl.ds(0, _ROWS)]],
                             rows_v.at[slot, pl.ds(h * _ROWS, _ROWS)], sem)

    def wait(chunk, slot):
        for h in range(2):
            pltpu.make_async_copy(
                table_hbm.at[fi_v.at[lax.rem(chunk, 3), h, pl.ds(0, _ROWS)]],
                rows_v.at[slot, pl.ds(h * _ROWS, _ROWS)], sem).wait()

    for c in range(2):
        prep_indices(c)
        issue(c, c)

    @pl.loop(0, _NCHUNK)
    def _chunk_body(chunk):
        slot = lax.rem(chunk, 3)
        wait(chunk, slot)

        @pl.when(chunk + 2 < _NCHUNK)
        def _():
            prep_indices(chunk + 2)
            issue(chunk + 2, lax.rem(chunk + 2, 3))

        buckets = []
        for b in range(_C):
            bidx = jnp.full((_L,), 0, jnp.int32) + (chunk * _C + b)
            plyb = plsc.load_gather(ply_v, [bidx])
            buckets.append(jnp.clip(plyb // _BUCKET_SIZE, 0, _NUM_LS_BUCKETS - 1))

        init = tuple(jnp.zeros((_L,), jnp.float32) for _ in range(_C))

        @pl.loop(0, _KCH, init_carry=init)
        def partials(k, carry):
            col = pl.ds(k * _L, _L)
            bias = ib_v[col]
            hbase = (k // 8) * _ROWS
            lcol = pl.ds(lax.rem(k, 8) * _L, _L)
            new = []
            for b in range(_C):
                base = b * _N_ACTIVE
                chains = [rows_v[slot, hbase + base + r, lcol]
                          for r in range(4)]
                for r in range(4, _N_ACTIVE):
                    chains[r % 4] = (chains[r % 4]
                                     + rows_v[slot, hbase + base + r, lcol])
                acc = (chains[0] + chains[1]) + (chains[2] + chains[3])
                acc = acc + bias
                acc = jnp.clip(acc, 0.0, 1.0)
                acc = acc * acc * jnp.float32(255.0 / 256.0)
                widx = buckets[b] * _LINPUT + (k * _L + lane)
                wch = plsc.load_gather(w_v, [widx])
                new.append(carry[b] + acc * wch)
            return tuple(new)

        vals = jnp.zeros((_L,), jnp.float32)
        for b in range(_C):
            rsum = jnp.sum(partials[b])
            ob = plsc.load_gather(ob_v, [buckets[b]])
            vals = vals + jnp.where(lane == b, rsum + ob, 0.0)

        out_v[pl.ds(chunk * _C, _L)] = vals

    pltpu.sync_copy(out_v.at[pl.ds(0, _BPW)],
                    out_hbm.at[pl.ds(wid * _BPW, _BPW)])


def kernel(feature_indices, mobility, ply, input_weight, input_bias,
           output_weight, output_bias):
    del mobility
    tbl = (input_weight.reshape(12500, 8, 2, _LN)
           .transpose(0, 2, 1, 3).reshape(_NLINES, _LN))
    w_flat = output_weight.reshape(-1)
    out = _sc_forward(feature_indices, ply.astype(jnp.int32), tbl, input_bias,
                      w_flat, output_bias)
    return out.reshape(_B, 1)


# speedup vs baseline: 1.7802x; 1.0221x over previous
import functools

import jax
import jax.numpy as jnp
from jax import lax
from jax.experimental import pallas as pl
from jax.experimental.pallas import tpu as pltpu
from jax.experimental.pallas import tpu_sc as plsc

_SUM_OF_FEATURES = 100000
_LINPUT = 256
_NUM_LS_BUCKETS = 60
_MAX_PLY = 60
_BUCKET_SIZE = _MAX_PLY // _NUM_LS_BUCKETS
_B = 16384
_N_ACTIVE = 26

_NC = 2
_NS = 16
_NW = _NC * _NS
_BPW = _B // _NW
_C = 4
_ROWS = _C * _N_ACTIVE
_RPAD = 112
_NCHUNK = _BPW // _C
_L = 16
_KCH = _LINPUT // _L

_LN = 128
_NLINES = _SUM_OF_FEATURES * 2

_mesh = plsc.VectorSubcoreMesh(
    core_axis_name="c", subcore_axis_name="s",
    num_cores=_NC, num_subcores=_NS)


@functools.partial(
    pl.kernel,
    out_type=jax.ShapeDtypeStruct((_B,), jnp.float32),
    mesh=_mesh,
    scratch_types=[
        pltpu.VMEM((_BPW * _N_ACTIVE // 128, 128), jnp.int32),
        pltpu.VMEM((3, 2, _RPAD), jnp.int32),
        pltpu.VMEM((_BPW,), jnp.int32),
        pltpu.VMEM((_NUM_LS_BUCKETS * _LINPUT,), jnp.float32),
        pltpu.VMEM((_LINPUT,), jnp.float32),
        pltpu.VMEM((_NUM_LS_BUCKETS,), jnp.float32),
        pltpu.VMEM((3, 2 * _ROWS, _LN), jnp.float32),
        pltpu.VMEM((_BPW + _L,), jnp.float32),
        pltpu.SemaphoreType.DMA,
    ],
    compiler_params=pltpu.CompilerParams(
        needs_layout_passes=False, use_tc_tiling_on_sc=False),
)
def _sc_forward(fi_hbm, ply_hbm, table_hbm, ib_hbm, w_hbm, ob_hbm, out_hbm,
                fi_raw_v, fi_v, ply_v, w_v, ib_v, ob_v, rows_v, out_v, sem):
    wid = lax.axis_index("s") * _NC + lax.axis_index("c")

    _FROWS = _BPW * _N_ACTIVE // 128
    pltpu.sync_copy(fi_hbm.at[pl.ds(wid * _FROWS, _FROWS)], fi_raw_v)
    pltpu.sync_copy(ply_hbm.at[pl.ds(wid * _BPW, _BPW)], ply_v)
    pltpu.sync_copy(w_hbm, w_v)
    pltpu.sync_copy(ib_hbm, ib_v)
    pltpu.sync_copy(ob_hbm, ob_v)

    lane = lax.iota(jnp.int32, _L)

    def prep_indices(chunk):
        slot = lax.rem(chunk, 3)
        for g in range(_RPAD // _L):
            q = jnp.minimum(chunk * _ROWS + (lane + g * _L),
                            _BPW * _N_ACTIVE - 1)
            v = plsc.load_gather(fi_raw_v, [q >> 7, q & 127])
            l0 = (v >> 3) * 16 + (v & 7)
            fi_v[slot, 0, pl.ds(g * _L, _L)] = l0
            fi_v[slot, 1, pl.ds(g * _L, _L)] = l0 + 8

    def issue(chunk, slot):
        islot = lax.rem(chunk, 3)
        for h in range(2):
            pltpu.async_copy(table_hbm.at[fi_v.at[islot, h, pl.ds(0, _ROWS)]],
                             rows_v.at[slot, pl.ds(h * _ROWS, _ROWS)], sem)

    def wait(chunk, slot):
        for h in range(2):
            pltpu.make_async_copy(
                table_hbm.at[fi_v.at[lax.rem(chunk, 3), h, pl.ds(0, _ROWS)]],
                rows_v.at[slot, pl.ds(h * _ROWS, _ROWS)], sem).wait()

    for c in range(2):
        prep_indices(c)
        issue(c, c)

    @pl.loop(0, _NCHUNK)
    def _chunk_body(chunk):
        slot = lax.rem(chunk, 3)
        wait(chunk, slot)

        @pl.when(chunk + 2 < _NCHUNK)
        def _():
            prep_indices(chunk + 2)
            issue(chunk + 2, lax.rem(chunk + 2, 3))

        buckets = []
        for b in range(_C):
            bidx = jnp.full((_L,), 0, jnp.int32) + (chunk * _C + b)
            plyb = plsc.load_gather(ply_v, [bidx])
            buckets.append(jnp.clip(plyb // _BUCKET_SIZE, 0, _NUM_LS_BUCKETS - 1))

        init = tuple(jnp.zeros((_L,), jnp.float32) for _ in range(_C))

        @pl.loop(0, _KCH, init_carry=init)
        def partials(k, carry):
            col = pl.ds(k * _L, _L)
            bias = ib_v[col]
            hbase = (k // 8) * _ROWS
            lcol = pl.ds(lax.rem(k, 8) * _L, _L)
            new = []
            for b in range(_C):
                base = b * _N_ACTIVE
                chains = [rows_v[slot, hbase + base + r, lcol]
                          for r in range(4)]
                for r in range(4, _N_ACTIVE):
                    chains[r % 4] = (chains[r % 4]
                                     + rows_v[slot, hbase + base + r, lcol])
                acc = (chains[0] + chains[1]) + (chains[2] + chains[3])
                acc = acc + bias
                acc = jnp.clip(acc, 0.0, 1.0)
                acc = acc * acc * jnp.float32(255.0 / 256.0)
                widx = buckets[b] * _LINPUT + (k * _L + lane)
                wch = plsc.load_gather(w_v, [widx])
                new.append(carry[b] + acc * wch)
            return tuple(new)

        vals = jnp.zeros((_L,), jnp.float32)
        for b in range(_C):
            rsum = jnp.sum(partials[b])
            ob = plsc.load_gather(ob_v, [buckets[b]])
            vals = vals + jnp.where(lane == b, rsum + ob, 0.0)

        out_v[pl.ds(chunk * _C, _L)] = vals

    pltpu.sync_copy(out_v.at[pl.ds(0, _BPW)],
                    out_hbm.at[pl.ds(wid * _BPW, _BPW)])


def kernel(feature_indices, mobility, ply, input_weight, input_bias,
           output_weight, output_bias):
    del mobility
    tbl = (input_weight.reshape(12500, 8, 2, _LN)
           .transpose(0, 2, 1, 3).reshape(_NLINES, _LN))
    w_flat = output_weight.reshape(-1)
    fi2 = feature_indices.reshape(_B * _N_ACTIVE // 128, 128)
    out = _sc_forward(fi2, ply.astype(jnp.int32), tbl, input_bias,
                      w_flat, output_bias)
    return out.reshape(_B, 1)
